# Initial kernel scaffold; baseline (speedup 1.0000x reference)
import jax
import jax.numpy as jnp
from jax.experimental import pallas as pl


def kernel(x, edge_index, batch, W1, b1, W2, b2, W3, b3, Wf1, bf1, Wf2, bf2, Wf3, bf3):
    raise NotImplementedError("write your pallas kernel here")


# speedup vs baseline: 11.7477x; 11.7477x over previous
import functools

import jax
import jax.numpy as jnp
from jax import lax
from jax.experimental import pallas as pl
from jax.experimental.pallas import tpu as pltpu
from jax.experimental.pallas import tpu_sc as plsc

N = 10000
E = 160000
D = 256
B = 8

NC = 2
NS = 16
CH = 125
NROWS = E // CH
RPS = NROWS // NS
RPW = NROWS // (NS * NC)
NPAD = 10240
NPS = NPAD // NS
NLAST = N - (NS - 1) * NPS
DEGW = 16

R = 400
NP = N // R

def _deg_body(dst_hbm, ones_hbm, zeros_hbm, out_hbm, idx_v, ones_v, acc, sem):
    c = lax.axis_index("c")
    s = lax.axis_index("s")
    pltpu.sync_copy(zeros_hbm, acc.at[pl.ds(s * NPS, NPS)])
    wid = s * NC + c
    pltpu.sync_copy(dst_hbm.at[pl.ds(wid * RPW, RPW)], idx_v)
    pltpu.sync_copy(ones_hbm, ones_v)
    plsc.subcore_barrier()

    def chunk(j, carry):
        pltpu.sync_copy(ones_v, acc.at[idx_v.at[j]], add=True)
        return carry

    lax.fori_loop(0, RPW, chunk, 0)
    plsc.subcore_barrier()
    _copy_out(acc, out_hbm, c, s)


def _copy_out(acc, out_hbm, c, s):
    @pl.when(s < NS - 1)
    def _():
        pltpu.sync_copy(acc.at[pl.ds(s * NPS, NPS)],
                        out_hbm.at[pl.ds(c * N + s * NPS, NPS)])

    @pl.when(s == NS - 1)
    def _():
        pltpu.sync_copy(acc.at[pl.ds((NS - 1) * NPS, NLAST)],
                        out_hbm.at[pl.ds(c * N + (NS - 1) * NPS, NLAST)])


@functools.cache
def _mesh():
    return plsc.VectorSubcoreMesh(core_axis_name="c", subcore_axis_name="s",
                                  num_cores=NC, num_subcores=NS)


_SC_PARAMS = pltpu.CompilerParams(use_tc_tiling_on_sc=False)


@functools.cache
def _deg_kernel():
    return pl.kernel(
        _deg_body,
        out_type=jax.ShapeDtypeStruct((NC * N, DEGW), jnp.float32),
        mesh=_mesh(),
        scratch_types=[
            pltpu.VMEM((RPW, CH), jnp.int32),
            pltpu.VMEM((CH, DEGW), jnp.float32),
            pltpu.VMEM_SHARED((NPAD, DEGW), jnp.float32),
            pltpu.SemaphoreType.DMA,
        ],
        compiler_params=_SC_PARAMS,
    )


def _edge_body(fh, srcb_hbm, dst_hbm, hcat_hbm, zeros_hbm, out_hbm,
               src_v, dst_v, rows, acc, sem):
    c = lax.axis_index("c")
    s = lax.axis_index("s")
    pltpu.sync_copy(zeros_hbm, acc.at[pl.ds(s * NPS, NPS)])
    pltpu.sync_copy(srcb_hbm.at[c].at[pl.ds(s * RPS, RPS)], src_v)
    pltpu.sync_copy(dst_hbm.at[pl.ds(s * RPS, RPS)], dst_v)
    plsc.subcore_barrier()

    def chunk(j, carry):
        pltpu.async_copy(hcat_hbm.at[src_v.at[j]], rows, sem).wait()
        pltpu.sync_copy(rows, acc.at[dst_v.at[j]], add=True)
        return carry

    lax.fori_loop(0, RPS, chunk, 0)
    plsc.subcore_barrier()
    _copy_out(acc, out_hbm, c, s)


@functools.cache
def _edge_kernel(fh):
    return pl.kernel(
        functools.partial(_edge_body, fh),
        out_type=jax.ShapeDtypeStruct((NC * N, fh), jnp.float32),
        mesh=_mesh(),
        scratch_types=[
            pltpu.VMEM((RPS, CH), jnp.int32),
            pltpu.VMEM((RPS, CH), jnp.int32),
            pltpu.VMEM((CH, fh), jnp.float32),
            pltpu.VMEM_SHARED((NPAD, fh), jnp.float32),
            pltpu.SemaphoreType.DMA,
        ],
        compiler_params=_SC_PARAMS,
    )


def _tc_first_body(x_ref, w_ref, degp_ref, dinv_ref, o_ref):
    deg = degp_ref[0, :, 0:1] + degp_ref[1, :, 0:1] + 1.0
    dinv = lax.rsqrt(deg)
    dinv_ref[...] = dinv
    u = jnp.dot(x_ref[...], w_ref[...], preferred_element_type=jnp.float32)
    u = u * dinv
    o_ref[0] = u[:, :32]
    o_ref[1] = u[:, 32:]


_tc_first = pl.pallas_call(
    _tc_first_body,
    grid=(NP,),
    in_specs=[
        pl.BlockSpec((R, D), lambda i: (i, 0)),
        pl.BlockSpec((D, 64), lambda i: (0, 0)),
        pl.BlockSpec((NC, R, DEGW), lambda i: (0, i, 0)),
    ],
    out_specs=[
        pl.BlockSpec((R, 1), lambda i: (i, 0)),
        pl.BlockSpec((NC, R, 32), lambda i: (0, i, 0)),
    ],
    out_shape=[
        jax.ShapeDtypeStruct((N, 1), jnp.float32),
        jax.ShapeDtypeStruct((NC, N, 32), jnp.float32),
    ],
)


def _tc_mid_body(hout, s_ref, h_ref, dinv_ref, b_ref, w_ref, o_ref):
    pre = jnp.concatenate([s_ref[0] + h_ref[0], s_ref[1] + h_ref[1]], axis=1)
    dinv = dinv_ref[...]
    h = jnp.maximum(dinv * pre + b_ref[...], 0.0)
    u = jnp.dot(h, w_ref[...], preferred_element_type=jnp.float32) * dinv
    o_ref[0] = u[:, :hout]
    o_ref[1] = u[:, hout:]


def _make_mid(din, dout):
    hin = din // 2
    hout = dout // 2
    return pl.pallas_call(
        functools.partial(_tc_mid_body, hout),
        grid=(NP,),
        in_specs=[
            pl.BlockSpec((NC, R, hin), lambda i: (0, i, 0)),
            pl.BlockSpec((NC, R, hin), lambda i: (0, i, 0)),
            pl.BlockSpec((R, 1), lambda i: (i, 0)),
            pl.BlockSpec((1, din), lambda i: (0, 0)),
            pl.BlockSpec((din, dout), lambda i: (0, 0)),
        ],
        out_specs=[pl.BlockSpec((NC, R, hout), lambda i: (0, i, 0))],
        out_shape=[jax.ShapeDtypeStruct((NC, N, hout), jnp.float32)],
    )


_tc_mid2 = _make_mid(64, 128)
_tc_mid3 = _make_mid(128, 256)


def _tc_head_body(s_ref, h_ref, dinv_ref, b3_ref, batch_ref,
                  wf1_ref, bf1_ref, wf2_ref, bf2_ref, wf3_ref, bf3_ref,
                  o_ref, acc, cnt):
    i = pl.program_id(0)

    @pl.when(i == 0)
    def _():
        acc[...] = jnp.zeros_like(acc)
        cnt[...] = jnp.zeros_like(cnt)

    pre = jnp.concatenate([s_ref[0] + h_ref[0], s_ref[1] + h_ref[1]], axis=1)
    h4 = jnp.maximum(dinv_ref[...] * pre + b3_ref[...], 0.0)
    cols = lax.broadcasted_iota(jnp.int32, (R, B), 1)
    maskt = (batch_ref[...] == cols).astype(jnp.float32)
    acc[...] += lax.dot_general(maskt, h4, (((0,), (0,)), ((), ())),
                                preferred_element_type=jnp.float32)
    cnt[...] += lax.dot_general(maskt, jnp.ones((R, 128), jnp.float32),
                                (((0,), (0,)), ((), ())),
                                preferred_element_type=jnp.float32)

    @pl.when(i == NP - 1)
    def _():
        g = acc[...] / jnp.maximum(cnt[:, 0:1], 1.0)
        g1 = jnp.maximum(
            jnp.dot(g, wf1_ref[...], preferred_element_type=jnp.float32)
            + bf1_ref[...], 0.0)
        g2 = jnp.maximum(
            jnp.dot(g1, wf2_ref[...], preferred_element_type=jnp.float32)
            + bf2_ref[...], 0.0)
        o_ref[...] = (jnp.dot(g2, wf3_ref[...],
                              preferred_element_type=jnp.float32)
                      + bf3_ref[...])


_tc_head = pl.pallas_call(
    _tc_head_body,
    grid=(NP,),
    in_specs=[
        pl.BlockSpec((NC, R, 128), lambda i: (0, i, 0)),
        pl.BlockSpec((NC, R, 128), lambda i: (0, i, 0)),
        pl.BlockSpec((R, 1), lambda i: (i, 0)),
        pl.BlockSpec((1, D), lambda i: (0, 0)),
        pl.BlockSpec((R, 1), lambda i: (i, 0)),
        pl.BlockSpec((D, 128), lambda i: (0, 0)),
        pl.BlockSpec((1, 128), lambda i: (0, 0)),
        pl.BlockSpec((128, 64), lambda i: (0, 0)),
        pl.BlockSpec((1, 64), lambda i: (0, 0)),
        pl.BlockSpec((64, 128), lambda i: (0, 0)),
        pl.BlockSpec((1, 128), lambda i: (0, 0)),
    ],
    out_specs=pl.BlockSpec((B, 128), lambda i: (0, 0)),
    out_shape=jax.ShapeDtypeStruct((B, 128), jnp.float32),
    scratch_shapes=[
        pltpu.VMEM((B, D), jnp.float32),
        pltpu.VMEM((B, 128), jnp.float32),
    ],
)


def kernel(x, edge_index, batch, W1, b1, W2, b2, W3, b3,
           Wf1, bf1, Wf2, bf2, Wf3, bf3):
    src = edge_index[0].reshape(NROWS, CH)
    dst = edge_index[1].reshape(NROWS, CH)
    srcb = jnp.stack([src, src + N])
    batch2 = batch.reshape(N, 1)

    ones16 = jnp.ones((CH, DEGW), jnp.float32)
    zeros16 = jnp.zeros((NPS, DEGW), jnp.float32)
    z32 = jnp.zeros((NPS, 32), jnp.float32)
    z64 = jnp.zeros((NPS, 64), jnp.float32)
    z128 = jnp.zeros((NPS, 128), jnp.float32)

    degp = _deg_kernel()(dst, ones16, zeros16).reshape(NC, N, DEGW)

    dinv, h1 = _tc_first(x, W1, degp)
    s1 = _edge_kernel(32)(srcb, dst, h1.reshape(NC * N, 32), z32).reshape(NC, N, 32)

    (h2,) = _tc_mid2(s1, h1, dinv, b1.reshape(1, 64), W2)
    s2 = _edge_kernel(64)(srcb, dst, h2.reshape(NC * N, 64), z64).reshape(NC, N, 64)

    (h3,) = _tc_mid3(s2, h2, dinv, b2.reshape(1, 128), W3)
    s3 = _edge_kernel(128)(srcb, dst, h3.reshape(NC * N, 128), z128).reshape(NC, N, 128)

    wf3p = jnp.pad(Wf3, ((0, 0), (0, 128 - Wf3.shape[1])))
    bf3p = jnp.pad(bf3, (0, 128 - bf3.shape[0])).reshape(1, 128)
    out = _tc_head(s3, h3, dinv, b3.reshape(1, D), batch2,
                   Wf1, bf1.reshape(1, 128), Wf2, bf2.reshape(1, 64),
                   wf3p, bf3p)
    return out[:, :Wf3.shape[1]]


# speedup vs baseline: 15.9492x; 1.3576x over previous
import functools

import jax
import jax.numpy as jnp
from jax import lax
from jax.experimental import pallas as pl
from jax.experimental.pallas import tpu as pltpu
from jax.experimental.pallas import tpu_sc as plsc

N = 10000
E = 160000
D = 256
B = 8

NC = 2
NS = 16
CH = 125
NROWS = E // CH
RPS = NROWS // NS
SEG = 2
SROWS = RPS // SEG
RPW = NROWS // (NS * NC)
NPAD = 10240
NPS = NPAD // NS
NLAST = N - (NS - 1) * NPS
DEGW = 16

R = 400
NP = N // R

def _deg_body(dst_hbm, ones_hbm, zeros_hbm, out_hbm, idx_v, ones_v, acc, sem):
    c = lax.axis_index("c")
    s = lax.axis_index("s")
    pltpu.sync_copy(zeros_hbm, acc.at[pl.ds(s * NPS, NPS)])
    wid = s * NC + c
    pltpu.sync_copy(dst_hbm.at[pl.ds(wid * RPW, RPW)], idx_v)
    pltpu.sync_copy(ones_hbm, ones_v)
    plsc.subcore_barrier()

    def chunk(j, carry):
        pltpu.sync_copy(ones_v, acc.at[idx_v.at[j]], add=True)
        return carry

    lax.fori_loop(0, RPW, chunk, 0)
    plsc.subcore_barrier()
    _copy_out(acc, out_hbm, c, s)


def _copy_out(acc, out_hbm, c, s):
    @pl.when(s < NS - 1)
    def _():
        pltpu.sync_copy(acc.at[pl.ds(s * NPS, NPS)],
                        out_hbm.at[pl.ds(c * N + s * NPS, NPS)])

    @pl.when(s == NS - 1)
    def _():
        pltpu.sync_copy(acc.at[pl.ds((NS - 1) * NPS, NLAST)],
                        out_hbm.at[pl.ds(c * N + (NS - 1) * NPS, NLAST)])


@functools.cache
def _mesh():
    return plsc.VectorSubcoreMesh(core_axis_name="c", subcore_axis_name="s",
                                  num_cores=NC, num_subcores=NS)


_SC_PARAMS = pltpu.CompilerParams(use_tc_tiling_on_sc=False)


@functools.cache
def _deg_kernel():
    return pl.kernel(
        _deg_body,
        out_type=jax.ShapeDtypeStruct((NC * N, DEGW), jnp.float32),
        mesh=_mesh(),
        scratch_types=[
            pltpu.VMEM((RPW, CH), jnp.int32),
            pltpu.VMEM((CH, DEGW), jnp.float32),
            pltpu.VMEM_SHARED((NPAD, DEGW), jnp.float32),
            pltpu.SemaphoreType.DMA,
        ],
        compiler_params=_SC_PARAMS,
    )


def _edge_body(fh, srcb_hbm, dst_hbm, hcat_hbm, zeros_hbm, out_hbm,
               src_v, dst_v, rows0, rows1, acc, sem0, sem1):
    c = lax.axis_index("c")
    s = lax.axis_index("s")
    pltpu.sync_copy(zeros_hbm, acc.at[pl.ds(s * NPS, NPS)])
    plsc.subcore_barrier()

    def seg(t, carry):
        base = s * RPS + t * SROWS
        pltpu.sync_copy(srcb_hbm.at[c].at[pl.ds(base, SROWS)], src_v)
        pltpu.sync_copy(dst_hbm.at[pl.ds(base, SROWS)], dst_v)
        pltpu.async_copy(hcat_hbm.at[src_v.at[0]], rows0, sem0)

        def pair(i, carry2):
            g = 2 * i
            d1 = pltpu.async_copy(hcat_hbm.at[src_v.at[g + 1]], rows1, sem1)
            pltpu.make_async_copy(hcat_hbm.at[src_v.at[g]], rows0, sem0).wait()
            pltpu.sync_copy(rows0, acc.at[dst_v.at[g]], add=True)

            @pl.when(g + 2 < SROWS)
            def _():
                pltpu.async_copy(hcat_hbm.at[src_v.at[g + 2]], rows0, sem0)

            d1.wait()
            pltpu.sync_copy(rows1, acc.at[dst_v.at[g + 1]], add=True)
            return carry2

        lax.fori_loop(0, SROWS // 2, pair, 0)
        return carry

    lax.fori_loop(0, SEG, seg, 0)
    plsc.subcore_barrier()
    _copy_out(acc, out_hbm, c, s)


@functools.cache
def _edge_kernel(fh):
    return pl.kernel(
        functools.partial(_edge_body, fh),
        out_type=jax.ShapeDtypeStruct((NC * N, fh), jnp.float32),
        mesh=_mesh(),
        scratch_types=[
            pltpu.VMEM((SROWS, CH), jnp.int32),
            pltpu.VMEM((SROWS, CH), jnp.int32),
            pltpu.VMEM((CH, fh), jnp.float32),
            pltpu.VMEM((CH, fh), jnp.float32),
            pltpu.VMEM_SHARED((NPAD, fh), jnp.float32),
            pltpu.SemaphoreType.DMA,
            pltpu.SemaphoreType.DMA,
        ],
        compiler_params=_SC_PARAMS,
    )


def _tc_first_body(x_ref, w_ref, degp_ref, dinv_ref, o_ref):
    deg = degp_ref[0, :, 0:1] + degp_ref[1, :, 0:1] + 1.0
    dinv = lax.rsqrt(deg)
    dinv_ref[...] = dinv
    u = jnp.dot(x_ref[...], w_ref[...], preferred_element_type=jnp.float32)
    u = u * dinv
    o_ref[0] = u[:, :32]
    o_ref[1] = u[:, 32:]


_tc_first = pl.pallas_call(
    _tc_first_body,
    grid=(NP,),
    in_specs=[
        pl.BlockSpec((R, D), lambda i: (i, 0)),
        pl.BlockSpec((D, 64), lambda i: (0, 0)),
        pl.BlockSpec((NC, R, DEGW), lambda i: (0, i, 0)),
    ],
    out_specs=[
        pl.BlockSpec((R, 1), lambda i: (i, 0)),
        pl.BlockSpec((NC, R, 32), lambda i: (0, i, 0)),
    ],
    out_shape=[
        jax.ShapeDtypeStruct((N, 1), jnp.float32),
        jax.ShapeDtypeStruct((NC, N, 32), jnp.float32),
    ],
)


def _tc_mid_body(hout, s_ref, h_ref, dinv_ref, b_ref, w_ref, o_ref):
    pre = jnp.concatenate([s_ref[0] + h_ref[0], s_ref[1] + h_ref[1]], axis=1)
    dinv = dinv_ref[...]
    h = jnp.maximum(dinv * pre + b_ref[...], 0.0)
    u = jnp.dot(h, w_ref[...], preferred_element_type=jnp.float32) * dinv
    o_ref[0] = u[:, :hout]
    o_ref[1] = u[:, hout:]


def _make_mid(din, dout):
    hin = din // 2
    hout = dout // 2
    return pl.pallas_call(
        functools.partial(_tc_mid_body, hout),
        grid=(NP,),
        in_specs=[
            pl.BlockSpec((NC, R, hin), lambda i: (0, i, 0)),
            pl.BlockSpec((NC, R, hin), lambda i: (0, i, 0)),
            pl.BlockSpec((R, 1), lambda i: (i, 0)),
            pl.BlockSpec((1, din), lambda i: (0, 0)),
            pl.BlockSpec((din, dout), lambda i: (0, 0)),
        ],
        out_specs=[pl.BlockSpec((NC, R, hout), lambda i: (0, i, 0))],
        out_shape=[jax.ShapeDtypeStruct((NC, N, hout), jnp.float32)],
    )


_tc_mid2 = _make_mid(64, 128)
_tc_mid3 = _make_mid(128, 256)


def _tc_head_body(s_ref, h_ref, dinv_ref, b3_ref, batch_ref,
                  wf1_ref, bf1_ref, wf2_ref, bf2_ref, wf3_ref, bf3_ref,
                  o_ref, acc, cnt):
    i = pl.program_id(0)

    @pl.when(i == 0)
    def _():
        acc[...] = jnp.zeros_like(acc)
        cnt[...] = jnp.zeros_like(cnt)

    pre = jnp.concatenate([s_ref[0] + h_ref[0], s_ref[1] + h_ref[1]], axis=1)
    h4 = jnp.maximum(dinv_ref[...] * pre + b3_ref[...], 0.0)
    cols = lax.broadcasted_iota(jnp.int32, (R, B), 1)
    maskt = (batch_ref[...] == cols).astype(jnp.float32)
    acc[...] += lax.dot_general(maskt, h4, (((0,), (0,)), ((), ())),
                                preferred_element_type=jnp.float32)
    cnt[...] += lax.dot_general(maskt, jnp.ones((R, 128), jnp.float32),
                                (((0,), (0,)), ((), ())),
                                preferred_element_type=jnp.float32)

    @pl.when(i == NP - 1)
    def _():
        g = acc[...] / jnp.maximum(cnt[:, 0:1], 1.0)
        g1 = jnp.maximum(
            jnp.dot(g, wf1_ref[...], preferred_element_type=jnp.float32)
            + bf1_ref[...], 0.0)
        g2 = jnp.maximum(
            jnp.dot(g1, wf2_ref[...], preferred_element_type=jnp.float32)
            + bf2_ref[...], 0.0)
        o_ref[...] = (jnp.dot(g2, wf3_ref[...],
                              preferred_element_type=jnp.float32)
                      + bf3_ref[...])


_tc_head = pl.pallas_call(
    _tc_head_body,
    grid=(NP,),
    in_specs=[
        pl.BlockSpec((NC, R, 128), lambda i: (0, i, 0)),
        pl.BlockSpec((NC, R, 128), lambda i: (0, i, 0)),
        pl.BlockSpec((R, 1), lambda i: (i, 0)),
        pl.BlockSpec((1, D), lambda i: (0, 0)),
        pl.BlockSpec((R, 1), lambda i: (i, 0)),
        pl.BlockSpec((D, 128), lambda i: (0, 0)),
        pl.BlockSpec((1, 128), lambda i: (0, 0)),
        pl.BlockSpec((128, 64), lambda i: (0, 0)),
        pl.BlockSpec((1, 64), lambda i: (0, 0)),
        pl.BlockSpec((64, 128), lambda i: (0, 0)),
        pl.BlockSpec((1, 128), lambda i: (0, 0)),
    ],
    out_specs=pl.BlockSpec((B, 128), lambda i: (0, 0)),
    out_shape=jax.ShapeDtypeStruct((B, 128), jnp.float32),
    scratch_shapes=[
        pltpu.VMEM((B, D), jnp.float32),
        pltpu.VMEM((B, 128), jnp.float32),
    ],
)


def kernel(x, edge_index, batch, W1, b1, W2, b2, W3, b3,
           Wf1, bf1, Wf2, bf2, Wf3, bf3):
    src = edge_index[0].reshape(NROWS, CH)
    dst = edge_index[1].reshape(NROWS, CH)
    srcb = jnp.stack([src, src + N])
    batch2 = batch.reshape(N, 1)

    ones16 = jnp.ones((CH, DEGW), jnp.float32)
    zeros16 = jnp.zeros((NPS, DEGW), jnp.float32)
    z32 = jnp.zeros((NPS, 32), jnp.float32)
    z64 = jnp.zeros((NPS, 64), jnp.float32)
    z128 = jnp.zeros((NPS, 128), jnp.float32)

    degp = _deg_kernel()(dst, ones16, zeros16).reshape(NC, N, DEGW)

    dinv, h1 = _tc_first(x, W1, degp)
    s1 = _edge_kernel(32)(srcb, dst, h1.reshape(NC * N, 32), z32).reshape(NC, N, 32)

    (h2,) = _tc_mid2(s1, h1, dinv, b1.reshape(1, 64), W2)
    s2 = _edge_kernel(64)(srcb, dst, h2.reshape(NC * N, 64), z64).reshape(NC, N, 64)

    (h3,) = _tc_mid3(s2, h2, dinv, b2.reshape(1, 128), W3)
    s3 = _edge_kernel(128)(srcb, dst, h3.reshape(NC * N, 128), z128).reshape(NC, N, 128)

    wf3p = jnp.pad(Wf3, ((0, 0), (0, 128 - Wf3.shape[1])))
    bf3p = jnp.pad(bf3, (0, 128 - bf3.shape[0])).reshape(1, 128)
    out = _tc_head(s3, h3, dinv, b3.reshape(1, D), batch2,
                   Wf1, bf1.reshape(1, 128), Wf2, bf2.reshape(1, 64),
                   wf3p, bf3p)
    return out[:, :Wf3.shape[1]]


# speedup vs baseline: 15.9867x; 1.0024x over previous
import functools

import jax
import jax.numpy as jnp
from jax import lax
from jax.experimental import pallas as pl
from jax.experimental.pallas import tpu as pltpu
from jax.experimental.pallas import tpu_sc as plsc

N = 10000
E = 160000
D = 256
B = 8

NC = 2
NS = 16
CH = 125
NROWS = E // CH
RPS = NROWS // NS
SEG = 2
SROWS = RPS // SEG
RPW = NROWS // (NS * NC)
NPAD = 10240
NPS = NPAD // NS
NLAST = N - (NS - 1) * NPS
DEGW = 16

R = 400
NP = N // R

def _deg_body(dst_hbm, ones_hbm, zeros_hbm, out_hbm, idx_v, ones_v, acc, sem):
    c = lax.axis_index("c")
    s = lax.axis_index("s")
    pltpu.sync_copy(zeros_hbm, acc.at[pl.ds(s * NPS, NPS)])
    wid = s * NC + c
    pltpu.sync_copy(dst_hbm.at[pl.ds(wid * RPW, RPW)], idx_v)
    pltpu.sync_copy(ones_hbm, ones_v)
    plsc.subcore_barrier()

    def chunk(j, carry):
        pltpu.sync_copy(ones_v, acc.at[idx_v.at[j]], add=True)
        return carry

    lax.fori_loop(0, RPW, chunk, 0)
    plsc.subcore_barrier()
    _copy_out(acc, out_hbm, c, s)


def _copy_out(acc, out_hbm, c, s):
    @pl.when(s < NS - 1)
    def _():
        pltpu.sync_copy(acc.at[pl.ds(s * NPS, NPS)],
                        out_hbm.at[pl.ds(c * N + s * NPS, NPS)])

    @pl.when(s == NS - 1)
    def _():
        pltpu.sync_copy(acc.at[pl.ds((NS - 1) * NPS, NLAST)],
                        out_hbm.at[pl.ds(c * N + (NS - 1) * NPS, NLAST)])


@functools.cache
def _mesh():
    return plsc.VectorSubcoreMesh(core_axis_name="c", subcore_axis_name="s",
                                  num_cores=NC, num_subcores=NS)


_SC_PARAMS = pltpu.CompilerParams(use_tc_tiling_on_sc=False)


@functools.cache
def _deg_kernel():
    return pl.kernel(
        _deg_body,
        out_type=jax.ShapeDtypeStruct((NC * N, DEGW), jnp.float32),
        mesh=_mesh(),
        scratch_types=[
            pltpu.VMEM((RPW, CH), jnp.int32),
            pltpu.VMEM((CH, DEGW), jnp.float32),
            pltpu.VMEM_SHARED((NPAD, DEGW), jnp.float32),
            pltpu.SemaphoreType.DMA,
        ],
        compiler_params=_SC_PARAMS,
    )


def _edge_body(fh, srcb_hbm, dst_hbm, hcat_hbm, zeros_hbm, out_hbm,
               src_v, dst_v, rows0, rows1, acc, sem0, sem1):
    c = lax.axis_index("c")
    s = lax.axis_index("s")
    pltpu.sync_copy(zeros_hbm, acc.at[pl.ds(s * NPS, NPS)])
    plsc.subcore_barrier()

    def seg(t, carry):
        base = s * RPS + t * SROWS
        pltpu.sync_copy(srcb_hbm.at[c].at[pl.ds(base, SROWS)], src_v)
        pltpu.sync_copy(dst_hbm.at[pl.ds(base, SROWS)], dst_v)
        pltpu.async_copy(hcat_hbm.at[src_v.at[0]], rows0, sem0)

        def pair(i, carry2):
            g = 2 * i
            d1 = pltpu.async_copy(hcat_hbm.at[src_v.at[g + 1]], rows1, sem1)
            pltpu.make_async_copy(hcat_hbm.at[src_v.at[g]], rows0, sem0).wait()
            pltpu.sync_copy(rows0, acc.at[dst_v.at[g]], add=True)

            @pl.when(g + 2 < SROWS)
            def _():
                pltpu.async_copy(hcat_hbm.at[src_v.at[g + 2]], rows0, sem0)

            d1.wait()
            pltpu.sync_copy(rows1, acc.at[dst_v.at[g + 1]], add=True)
            return carry2

        lax.fori_loop(0, SROWS // 2, pair, 0)
        return carry

    lax.fori_loop(0, SEG, seg, 0)
    plsc.subcore_barrier()
    _copy_out(acc, out_hbm, c, s)


@functools.cache
def _edge_kernel(fh):
    return pl.kernel(
        functools.partial(_edge_body, fh),
        out_type=jax.ShapeDtypeStruct((NC * N, fh), jnp.float32),
        mesh=_mesh(),
        scratch_types=[
            pltpu.VMEM((SROWS, CH), jnp.int32),
            pltpu.VMEM((SROWS, CH), jnp.int32),
            pltpu.VMEM((CH, fh), jnp.float32),
            pltpu.VMEM((CH, fh), jnp.float32),
            pltpu.VMEM_SHARED((NPAD, fh), jnp.float32),
            pltpu.SemaphoreType.DMA,
            pltpu.SemaphoreType.DMA,
        ],
        compiler_params=None if fh == 128 else _SC_PARAMS,
    )


def _tc_first_body(x_ref, w_ref, degp_ref, dinv_ref, o_ref):
    deg = degp_ref[0, :, 0:1] + degp_ref[1, :, 0:1] + 1.0
    dinv = lax.rsqrt(deg)
    dinv_ref[...] = dinv
    u = jnp.dot(x_ref[...], w_ref[...], preferred_element_type=jnp.float32)
    u = u * dinv
    o_ref[0] = u[:, :32]
    o_ref[1] = u[:, 32:]


_tc_first = pl.pallas_call(
    _tc_first_body,
    grid=(NP,),
    in_specs=[
        pl.BlockSpec((R, D), lambda i: (i, 0)),
        pl.BlockSpec((D, 64), lambda i: (0, 0)),
        pl.BlockSpec((NC, R, DEGW), lambda i: (0, i, 0)),
    ],
    out_specs=[
        pl.BlockSpec((R, 1), lambda i: (i, 0)),
        pl.BlockSpec((NC, R, 32), lambda i: (0, i, 0)),
    ],
    out_shape=[
        jax.ShapeDtypeStruct((N, 1), jnp.float32),
        jax.ShapeDtypeStruct((NC, N, 32), jnp.float32),
    ],
)


def _tc_mid_body(hout, s_ref, h_ref, dinv_ref, b_ref, w_ref, o_ref):
    pre = jnp.concatenate([s_ref[0] + h_ref[0], s_ref[1] + h_ref[1]], axis=1)
    dinv = dinv_ref[...]
    h = jnp.maximum(dinv * pre + b_ref[...], 0.0)
    u = jnp.dot(h, w_ref[...], preferred_element_type=jnp.float32) * dinv
    o_ref[0] = u[:, :hout]
    o_ref[1] = u[:, hout:]


def _make_mid(din, dout):
    hin = din // 2
    hout = dout // 2
    return pl.pallas_call(
        functools.partial(_tc_mid_body, hout),
        grid=(NP,),
        in_specs=[
            pl.BlockSpec((NC, R, hin), lambda i: (0, i, 0)),
            pl.BlockSpec((NC, R, hin), lambda i: (0, i, 0)),
            pl.BlockSpec((R, 1), lambda i: (i, 0)),
            pl.BlockSpec((1, din), lambda i: (0, 0)),
            pl.BlockSpec((din, dout), lambda i: (0, 0)),
        ],
        out_specs=[pl.BlockSpec((NC, R, hout), lambda i: (0, i, 0))],
        out_shape=[jax.ShapeDtypeStruct((NC, N, hout), jnp.float32)],
    )


_tc_mid2 = _make_mid(64, 128)
_tc_mid3 = _make_mid(128, 256)


def _tc_head_body(s_ref, h_ref, dinv_ref, b3_ref, batch_ref,
                  wf1_ref, bf1_ref, wf2_ref, bf2_ref, wf3_ref, bf3_ref,
                  o_ref, acc, cnt):
    i = pl.program_id(0)

    @pl.when(i == 0)
    def _():
        acc[...] = jnp.zeros_like(acc)
        cnt[...] = jnp.zeros_like(cnt)

    pre = jnp.concatenate([s_ref[0] + h_ref[0], s_ref[1] + h_ref[1]], axis=1)
    h4 = jnp.maximum(dinv_ref[...] * pre + b3_ref[...], 0.0)
    cols = lax.broadcasted_iota(jnp.int32, (R, B), 1)
    maskt = (batch_ref[...] == cols).astype(jnp.float32)
    acc[...] += lax.dot_general(maskt, h4, (((0,), (0,)), ((), ())),
                                preferred_element_type=jnp.float32)
    cnt[...] += lax.dot_general(maskt, jnp.ones((R, 128), jnp.float32),
                                (((0,), (0,)), ((), ())),
                                preferred_element_type=jnp.float32)

    @pl.when(i == NP - 1)
    def _():
        g = acc[...] / jnp.maximum(cnt[:, 0:1], 1.0)
        g1 = jnp.maximum(
            jnp.dot(g, wf1_ref[...], preferred_element_type=jnp.float32)
            + bf1_ref[...], 0.0)
        g2 = jnp.maximum(
            jnp.dot(g1, wf2_ref[...], preferred_element_type=jnp.float32)
            + bf2_ref[...], 0.0)
        o_ref[...] = (jnp.dot(g2, wf3_ref[...],
                              preferred_element_type=jnp.float32)
                      + bf3_ref[...])


_tc_head = pl.pallas_call(
    _tc_head_body,
    grid=(NP,),
    in_specs=[
        pl.BlockSpec((NC, R, 128), lambda i: (0, i, 0)),
        pl.BlockSpec((NC, R, 128), lambda i: (0, i, 0)),
        pl.BlockSpec((R, 1), lambda i: (i, 0)),
        pl.BlockSpec((1, D), lambda i: (0, 0)),
        pl.BlockSpec((R, 1), lambda i: (i, 0)),
        pl.BlockSpec((D, 128), lambda i: (0, 0)),
        pl.BlockSpec((1, 128), lambda i: (0, 0)),
        pl.BlockSpec((128, 64), lambda i: (0, 0)),
        pl.BlockSpec((1, 64), lambda i: (0, 0)),
        pl.BlockSpec((64, 128), lambda i: (0, 0)),
        pl.BlockSpec((1, 128), lambda i: (0, 0)),
    ],
    out_specs=pl.BlockSpec((B, 128), lambda i: (0, 0)),
    out_shape=jax.ShapeDtypeStruct((B, 128), jnp.float32),
    scratch_shapes=[
        pltpu.VMEM((B, D), jnp.float32),
        pltpu.VMEM((B, 128), jnp.float32),
    ],
)


def kernel(x, edge_index, batch, W1, b1, W2, b2, W3, b3,
           Wf1, bf1, Wf2, bf2, Wf3, bf3):
    src = edge_index[0].reshape(NROWS, CH)
    dst = edge_index[1].reshape(NROWS, CH)
    srcb = jnp.stack([src, src + N])
    batch2 = batch.reshape(N, 1)

    ones16 = jnp.ones((CH, DEGW), jnp.float32)
    zeros16 = jnp.zeros((NPS, DEGW), jnp.float32)
    z32 = jnp.zeros((NPS, 32), jnp.float32)
    z64 = jnp.zeros((NPS, 64), jnp.float32)
    z128 = jnp.zeros((NPS, 128), jnp.float32)

    degp = _deg_kernel()(dst, ones16, zeros16).reshape(NC, N, DEGW)

    dinv, h1 = _tc_first(x, W1, degp)
    s1 = _edge_kernel(32)(srcb, dst, h1.reshape(NC * N, 32), z32).reshape(NC, N, 32)

    (h2,) = _tc_mid2(s1, h1, dinv, b1.reshape(1, 64), W2)
    s2 = _edge_kernel(64)(srcb, dst, h2.reshape(NC * N, 64), z64).reshape(NC, N, 64)

    (h3,) = _tc_mid3(s2, h2, dinv, b2.reshape(1, 128), W3)
    s3 = _edge_kernel(128)(srcb, dst, h3.reshape(NC * N, 128), z128).reshape(NC, N, 128)

    wf3p = jnp.pad(Wf3, ((0, 0), (0, 128 - Wf3.shape[1])))
    bf3p = jnp.pad(bf3, (0, 128 - bf3.shape[0])).reshape(1, 128)
    out = _tc_head(s3, h3, dinv, b3.reshape(1, D), batch2,
                   Wf1, bf1.reshape(1, 128), Wf2, bf2.reshape(1, 64),
                   wf3p, bf3p)
    return out[:, :Wf3.shape[1]]


# speedup vs baseline: 15.9928x; 1.0004x over previous
import functools

import jax
import jax.numpy as jnp
from jax import lax
from jax.experimental import pallas as pl
from jax.experimental.pallas import tpu as pltpu
from jax.experimental.pallas import tpu_sc as plsc

N = 10000
E = 160000
D = 256
B = 8

NC = 2
NS = 16
CH = 125
NROWS = E // CH
RPS = NROWS // NS
SEG = 2
SROWS = RPS // SEG
RPW = NROWS // (NS * NC)
NPAD = 10240
NPS = NPAD // NS
NLAST = N - (NS - 1) * NPS
DEGW = 16

R = 400
NP = N // R

def _deg_body(dst_hbm, ones_hbm, zeros_hbm, out_hbm, idx_v, ones_v, acc, sem):
    c = lax.axis_index("c")
    s = lax.axis_index("s")
    pltpu.sync_copy(zeros_hbm, acc.at[pl.ds(s * NPS, NPS)])
    wid = s * NC + c
    pltpu.sync_copy(dst_hbm.at[pl.ds(wid * RPW, RPW)], idx_v)
    pltpu.sync_copy(ones_hbm, ones_v)
    plsc.subcore_barrier()

    def chunk(j, carry):
        pltpu.sync_copy(ones_v, acc.at[idx_v.at[j]], add=True)
        return carry

    lax.fori_loop(0, RPW, chunk, 0)
    plsc.subcore_barrier()
    _copy_out(acc, out_hbm.at[c], s)


def _copy_out(acc, out2d, s):
    @pl.when(s < NS - 1)
    def _():
        pltpu.sync_copy(acc.at[pl.ds(s * NPS, NPS)],
                        out2d.at[pl.ds(s * NPS, NPS)])

    @pl.when(s == NS - 1)
    def _():
        pltpu.sync_copy(acc.at[pl.ds((NS - 1) * NPS, NLAST)],
                        out2d.at[pl.ds((NS - 1) * NPS, NLAST)])


@functools.cache
def _mesh():
    return plsc.VectorSubcoreMesh(core_axis_name="c", subcore_axis_name="s",
                                  num_cores=NC, num_subcores=NS)


_SC_PARAMS = pltpu.CompilerParams(use_tc_tiling_on_sc=False)


@functools.cache
def _deg_kernel():
    return pl.kernel(
        _deg_body,
        out_type=jax.ShapeDtypeStruct((NC, N, DEGW), jnp.float32),
        mesh=_mesh(),
        scratch_types=[
            pltpu.VMEM((RPW, CH), jnp.int32),
            pltpu.VMEM((CH, DEGW), jnp.float32),
            pltpu.VMEM_SHARED((NPAD, DEGW), jnp.float32),
            pltpu.SemaphoreType.DMA,
        ],
        compiler_params=_SC_PARAMS,
    )


def _edge_body(fh, src_hbm, dst_hbm, hcat_hbm, zeros_hbm, out_hbm,
               src_v, dst_v, rows0, rows1, acc, sem0, sem1):
    c = lax.axis_index("c")
    s = lax.axis_index("s")
    pltpu.sync_copy(zeros_hbm, acc.at[pl.ds(s * NPS, NPS)])
    tab = hcat_hbm.at[c]
    plsc.subcore_barrier()

    def seg(t, carry):
        base = s * RPS + t * SROWS
        pltpu.sync_copy(src_hbm.at[pl.ds(base, SROWS)], src_v)
        pltpu.sync_copy(dst_hbm.at[pl.ds(base, SROWS)], dst_v)
        pltpu.async_copy(tab.at[src_v.at[0]], rows0, sem0)

        def pair(i, carry2):
            g = 2 * i
            d1 = pltpu.async_copy(tab.at[src_v.at[g + 1]], rows1, sem1)
            pltpu.make_async_copy(tab.at[src_v.at[g]], rows0, sem0).wait()
            pltpu.sync_copy(rows0, acc.at[dst_v.at[g]], add=True)

            @pl.when(g + 2 < SROWS)
            def _():
                pltpu.async_copy(tab.at[src_v.at[g + 2]], rows0, sem0)

            d1.wait()
            pltpu.sync_copy(rows1, acc.at[dst_v.at[g + 1]], add=True)
            return carry2

        lax.fori_loop(0, SROWS // 2, pair, 0)
        return carry

    lax.fori_loop(0, SEG, seg, 0)
    plsc.subcore_barrier()
    _copy_out(acc, out_hbm.at[c], s)


@functools.cache
def _edge_kernel(fh):
    return pl.kernel(
        functools.partial(_edge_body, fh),
        out_type=jax.ShapeDtypeStruct((NC, N, fh), jnp.float32),
        mesh=_mesh(),
        scratch_types=[
            pltpu.VMEM((SROWS, CH), jnp.int32),
            pltpu.VMEM((SROWS, CH), jnp.int32),
            pltpu.VMEM((CH, fh), jnp.float32),
            pltpu.VMEM((CH, fh), jnp.float32),
            pltpu.VMEM_SHARED((NPAD, fh), jnp.float32),
            pltpu.SemaphoreType.DMA,
            pltpu.SemaphoreType.DMA,
        ],
        compiler_params=None if fh == 128 else _SC_PARAMS,
    )


def _tc_first_body(x_ref, w_ref, degp_ref, dinv_ref, o_ref):
    deg = degp_ref[0, :, 0:1] + degp_ref[1, :, 0:1] + 1.0
    dinv = lax.rsqrt(deg)
    dinv_ref[...] = dinv
    u = jnp.dot(x_ref[...], w_ref[...], preferred_element_type=jnp.float32)
    u = u * dinv
    o_ref[0] = u[:, :32]
    o_ref[1] = u[:, 32:]


_tc_first = pl.pallas_call(
    _tc_first_body,
    grid=(NP,),
    in_specs=[
        pl.BlockSpec((R, D), lambda i: (i, 0)),
        pl.BlockSpec((D, 64), lambda i: (0, 0)),
        pl.BlockSpec((NC, R, DEGW), lambda i: (0, i, 0)),
    ],
    out_specs=[
        pl.BlockSpec((R, 1), lambda i: (i, 0)),
        pl.BlockSpec((NC, R, 32), lambda i: (0, i, 0)),
    ],
    out_shape=[
        jax.ShapeDtypeStruct((N, 1), jnp.float32),
        jax.ShapeDtypeStruct((NC, N, 32), jnp.float32),
    ],
)


def _tc_mid_body(hout, s_ref, h_ref, dinv_ref, b_ref, w_ref, o_ref):
    pre = jnp.concatenate([s_ref[0] + h_ref[0], s_ref[1] + h_ref[1]], axis=1)
    dinv = dinv_ref[...]
    h = jnp.maximum(dinv * pre + b_ref[...], 0.0)
    u = jnp.dot(h, w_ref[...], preferred_element_type=jnp.float32) * dinv
    o_ref[0] = u[:, :hout]
    o_ref[1] = u[:, hout:]


def _make_mid(din, dout):
    hin = din // 2
    hout = dout // 2
    return pl.pallas_call(
        functools.partial(_tc_mid_body, hout),
        grid=(NP,),
        in_specs=[
            pl.BlockSpec((NC, R, hin), lambda i: (0, i, 0)),
            pl.BlockSpec((NC, R, hin), lambda i: (0, i, 0)),
            pl.BlockSpec((R, 1), lambda i: (i, 0)),
            pl.BlockSpec((1, din), lambda i: (0, 0)),
            pl.BlockSpec((din, dout), lambda i: (0, 0)),
        ],
        out_specs=[pl.BlockSpec((NC, R, hout), lambda i: (0, i, 0))],
        out_shape=[jax.ShapeDtypeStruct((NC, N, hout), jnp.float32)],
    )


_tc_mid2 = _make_mid(64, 128)
_tc_mid3 = _make_mid(128, 256)


def _tc_head_body(s_ref, h_ref, dinv_ref, b3_ref, batch_ref,
                  wf1_ref, bf1_ref, wf2_ref, bf2_ref, wf3_ref, bf3_ref,
                  o_ref, acc, cnt):
    i = pl.program_id(0)

    @pl.when(i == 0)
    def _():
        acc[...] = jnp.zeros_like(acc)
        cnt[...] = jnp.zeros_like(cnt)

    pre = jnp.concatenate([s_ref[0] + h_ref[0], s_ref[1] + h_ref[1]], axis=1)
    h4 = jnp.maximum(dinv_ref[...] * pre + b3_ref[...], 0.0)
    cols = lax.broadcasted_iota(jnp.int32, (R, B), 1)
    maskt = (batch_ref[...] == cols).astype(jnp.float32)
    acc[...] += lax.dot_general(maskt, h4, (((0,), (0,)), ((), ())),
                                preferred_element_type=jnp.float32)
    cnt[...] += lax.dot_general(maskt, jnp.ones((R, 128), jnp.float32),
                                (((0,), (0,)), ((), ())),
                                preferred_element_type=jnp.float32)

    @pl.when(i == NP - 1)
    def _():
        g = acc[...] / jnp.maximum(cnt[:, 0:1], 1.0)
        g1 = jnp.maximum(
            jnp.dot(g, wf1_ref[...], preferred_element_type=jnp.float32)
            + bf1_ref[...], 0.0)
        g2 = jnp.maximum(
            jnp.dot(g1, wf2_ref[...], preferred_element_type=jnp.float32)
            + bf2_ref[...], 0.0)
        o_ref[...] = (jnp.dot(g2, wf3_ref[...],
                              preferred_element_type=jnp.float32)
                      + bf3_ref[...])


_tc_head = pl.pallas_call(
    _tc_head_body,
    grid=(NP,),
    in_specs=[
        pl.BlockSpec((NC, R, 128), lambda i: (0, i, 0)),
        pl.BlockSpec((NC, R, 128), lambda i: (0, i, 0)),
        pl.BlockSpec((R, 1), lambda i: (i, 0)),
        pl.BlockSpec((1, D), lambda i: (0, 0)),
        pl.BlockSpec((R, 1), lambda i: (i, 0)),
        pl.BlockSpec((D, 128), lambda i: (0, 0)),
        pl.BlockSpec((1, 128), lambda i: (0, 0)),
        pl.BlockSpec((128, 64), lambda i: (0, 0)),
        pl.BlockSpec((1, 64), lambda i: (0, 0)),
        pl.BlockSpec((64, 128), lambda i: (0, 0)),
        pl.BlockSpec((1, 128), lambda i: (0, 0)),
    ],
    out_specs=pl.BlockSpec((B, 128), lambda i: (0, 0)),
    out_shape=jax.ShapeDtypeStruct((B, 128), jnp.float32),
    scratch_shapes=[
        pltpu.VMEM((B, D), jnp.float32),
        pltpu.VMEM((B, 128), jnp.float32),
    ],
)


def kernel(x, edge_index, batch, W1, b1, W2, b2, W3, b3,
           Wf1, bf1, Wf2, bf2, Wf3, bf3):
    src = edge_index[0].reshape(NROWS, CH)
    dst = edge_index[1].reshape(NROWS, CH)
    batch2 = batch.reshape(N, 1)

    ones16 = jnp.ones((CH, DEGW), jnp.float32)
    zeros16 = jnp.zeros((NPS, DEGW), jnp.float32)
    z32 = jnp.zeros((NPS, 32), jnp.float32)
    z64 = jnp.zeros((NPS, 64), jnp.float32)
    z128 = jnp.zeros((NPS, 128), jnp.float32)

    degp = _deg_kernel()(dst, ones16, zeros16)

    dinv, h1 = _tc_first(x, W1, degp)
    s1 = _edge_kernel(32)(src, dst, h1, z32)

    (h2,) = _tc_mid2(s1, h1, dinv, b1.reshape(1, 64), W2)
    s2 = _edge_kernel(64)(src, dst, h2, z64)

    (h3,) = _tc_mid3(s2, h2, dinv, b2.reshape(1, 128), W3)
    s3 = _edge_kernel(128)(src, dst, h3, z128)

    wf3p = jnp.pad(Wf3, ((0, 0), (0, 128 - Wf3.shape[1])))
    bf3p = jnp.pad(bf3, (0, 128 - bf3.shape[0])).reshape(1, 128)
    out = _tc_head(s3, h3, dinv, b3.reshape(1, D), batch2,
                   Wf1, bf1.reshape(1, 128), Wf2, bf2.reshape(1, 64),
                   wf3p, bf3p)
    return out[:, :Wf3.shape[1]]


# speedup vs baseline: 18.7938x; 1.1751x over previous
import functools

import jax
import jax.numpy as jnp
from jax import lax
from jax.experimental import pallas as pl
from jax.experimental.pallas import tpu as pltpu
from jax.experimental.pallas import tpu_sc as plsc

N = 10000
E = 160000
D = 256
B = 8

NC = 2
NS = 16
CH = 125
NROWS = E // CH
RPS = NROWS // NS
SEG = 2
SROWS = RPS // SEG
RPW = NROWS // (NS * NC)
NPAD = 10240
NPS = NPAD // NS
NLAST = N - (NS - 1) * NPS
DEGW = 32

NQ = N // 4
RQ = 500
NPQ = NQ // RQ
R = 400
NP = N // R


def _deg_body(e_hbm, ones_hbm, zeros_hbm, out_hbm, idx_v, ones_v, acc, sem):
    c = lax.axis_index("c")
    s = lax.axis_index("s")
    pltpu.sync_copy(zeros_hbm, acc.at[pl.ds(s * NPS, NPS)])
    wid = c * NS + s
    pltpu.sync_copy(e_hbm.at[1].at[pl.ds(wid * RPW, RPW)], idx_v)
    pltpu.sync_copy(ones_hbm, ones_v)
    plsc.subcore_barrier()

    def chunk(j, carry):
        pltpu.sync_copy(ones_v, acc.at[idx_v.at[j]], add=True)
        return carry

    lax.fori_loop(0, RPW, chunk, 0)
    plsc.subcore_barrier()
    _copy_out(acc, out_hbm.at[c], s)


def _copy_out(acc, out2d, s):
    @pl.when(s < NS - 1)
    def _():
        pltpu.sync_copy(acc.at[pl.ds(s * NPS, NPS)],
                        out2d.at[pl.ds(s * NPS, NPS)])

    @pl.when(s == NS - 1)
    def _():
        pltpu.sync_copy(acc.at[pl.ds((NS - 1) * NPS, NLAST)],
                        out2d.at[pl.ds((NS - 1) * NPS, NLAST)])


@functools.cache
def _mesh():
    return plsc.VectorSubcoreMesh(core_axis_name="c", subcore_axis_name="s",
                                  num_cores=NC, num_subcores=NS)


_SC_PARAMS = pltpu.CompilerParams(use_tc_tiling_on_sc=False)


@functools.cache
def _deg_kernel():
    return pl.kernel(
        _deg_body,
        out_type=jax.ShapeDtypeStruct((NC, N, DEGW), jnp.float32),
        mesh=_mesh(),
        scratch_types=[
            pltpu.VMEM((RPW, CH), jnp.int32),
            pltpu.VMEM((CH, DEGW), jnp.float32),
            pltpu.VMEM_SHARED((NPAD, DEGW), jnp.float32),
            pltpu.SemaphoreType.DMA,
        ],
        compiler_params=_SC_PARAMS,
    )


def _edge_pipeline(e_hbm, tab, acc, src_v, dst_v, rows0, rows1, sem0, sem1,
                   row_base, nrows, nseg):
    def seg(t, carry):
        base = row_base + t * nrows
        pltpu.sync_copy(e_hbm.at[0].at[pl.ds(base, nrows)], src_v)
        pltpu.sync_copy(e_hbm.at[1].at[pl.ds(base, nrows)], dst_v)
        pltpu.async_copy(tab.at[src_v.at[0]], rows0, sem0)

        def pair(i, carry2):
            g = 2 * i
            d1 = pltpu.async_copy(tab.at[src_v.at[g + 1]], rows1, sem1)
            pltpu.make_async_copy(tab.at[src_v.at[g]], rows0, sem0).wait()
            pltpu.sync_copy(rows0, acc.at[dst_v.at[g]], add=True)

            @pl.when(g + 2 < nrows)
            def _():
                pltpu.async_copy(tab.at[src_v.at[g + 2]], rows0, sem0)

            d1.wait()
            pltpu.sync_copy(rows1, acc.at[dst_v.at[g + 1]], add=True)
            return carry2

        lax.fori_loop(0, nrows // 2, pair, 0)
        return carry

    lax.fori_loop(0, nseg, seg, 0)


def _edge_fsplit_body(fh, e_hbm, hcat_hbm, zeros_hbm, out_hbm,
                      src_v, dst_v, rows0, rows1, acc, sem0, sem1):
    c = lax.axis_index("c")
    s = lax.axis_index("s")
    pltpu.sync_copy(zeros_hbm, acc.at[pl.ds(s * NPS, NPS)])
    plsc.subcore_barrier()
    _edge_pipeline(e_hbm, hcat_hbm.at[c], acc, src_v, dst_v, rows0, rows1,
                   sem0, sem1, s * RPS, SROWS, SEG)
    plsc.subcore_barrier()
    _copy_out(acc, out_hbm.at[c], s)


def _edge_esplit_body(e_hbm, h_hbm, zeros_hbm, out_hbm,
                      src_v, dst_v, rows0, rows1, acc, sem0, sem1):
    c = lax.axis_index("c")
    s = lax.axis_index("s")
    pltpu.sync_copy(zeros_hbm, acc.at[pl.ds(s * NPS, NPS)])
    plsc.subcore_barrier()
    _edge_pipeline(e_hbm, h_hbm, acc, src_v, dst_v, rows0, rows1,
                   sem0, sem1, (c * NS + s) * RPW, RPW, 1)
    plsc.subcore_barrier()
    _copy_out(acc, out_hbm.at[c], s)


@functools.cache
def _edge_fsplit(fh):
    return pl.kernel(
        functools.partial(_edge_fsplit_body, fh),
        out_type=jax.ShapeDtypeStruct((NC, N, fh), jnp.float32),
        mesh=_mesh(),
        scratch_types=[
            pltpu.VMEM((SROWS, CH), jnp.int32),
            pltpu.VMEM((SROWS, CH), jnp.int32),
            pltpu.VMEM((CH, fh), jnp.float32),
            pltpu.VMEM((CH, fh), jnp.float32),
            pltpu.VMEM_SHARED((NPAD, fh), jnp.float32),
            pltpu.SemaphoreType.DMA,
            pltpu.SemaphoreType.DMA,
        ],
        compiler_params=_SC_PARAMS,
    )


@functools.cache
def _edge_esplit():
    return pl.kernel(
        _edge_esplit_body,
        out_type=jax.ShapeDtypeStruct((NC, N, 128), jnp.float32),
        mesh=_mesh(),
        scratch_types=[
            pltpu.VMEM((RPW, CH), jnp.int32),
            pltpu.VMEM((RPW, CH), jnp.int32),
            pltpu.VMEM((CH, 128), jnp.float32),
            pltpu.VMEM((CH, 128), jnp.float32),
            pltpu.VMEM_SHARED((NPAD, 128), jnp.float32),
            pltpu.SemaphoreType.DMA,
            pltpu.SemaphoreType.DMA,
        ],
        compiler_params=_SC_PARAMS,
    )


def _tc_first_body(x4_ref, w0_ref, w1_ref, degp_ref, o_ref, dinvb_ref):
    d = degp_ref[0] + degp_ref[1] + 1.0
    dinv4 = lax.rsqrt(d)
    x4 = x4_ref[...]
    o_ref[0] = jnp.dot(x4, w0_ref[...],
                       preferred_element_type=jnp.float32) * dinv4
    o_ref[1] = jnp.dot(x4, w1_ref[...],
                       preferred_element_type=jnp.float32) * dinv4
    li = lax.broadcasted_iota(jnp.int32, (128, 512), 0)
    oi = lax.broadcasted_iota(jnp.int32, (128, 512), 1)
    m = ((li % 32 == 0) & (oi // 128 == li // 32)).astype(jnp.float32)
    dinvb_ref[...] = jnp.dot(dinv4, m, preferred_element_type=jnp.float32)


_tc_first = pl.pallas_call(
    _tc_first_body,
    out_shape=[
        jax.ShapeDtypeStruct((NC, NQ, 128), jnp.float32),
        jax.ShapeDtypeStruct((NQ, 512), jnp.float32),
    ],
)


def _tc_mid2_body(s_ref, h_ref, degp_ref, b0_ref, b1_ref, w0_ref, w1_ref,
                  o_ref):
    d = degp_ref[0] + degp_ref[1] + 1.0
    dinv4 = lax.rsqrt(d)
    hp0 = jnp.maximum(dinv4 * (s_ref[0] + h_ref[0]) + b0_ref[...], 0.0) * dinv4
    hp1 = jnp.maximum(dinv4 * (s_ref[1] + h_ref[1]) + b1_ref[...], 0.0) * dinv4
    o_ref[...] = (
        jnp.dot(hp0, w0_ref[...], preferred_element_type=jnp.float32)
        + jnp.dot(hp1, w1_ref[...], preferred_element_type=jnp.float32))


_tc_mid2 = pl.pallas_call(
    _tc_mid2_body,
    out_shape=jax.ShapeDtypeStruct((NQ, 512), jnp.float32),
)


def _tc_mid3_body(s_ref, h_ref, dinvb_ref, b_ref, w_ref, o_ref):
    dinv = dinvb_ref[...]
    pre = s_ref[0] + s_ref[1] + h_ref[...]
    h = jnp.maximum(dinv * pre + b_ref[...], 0.0) * dinv
    u = jnp.dot(h, w_ref[...], preferred_element_type=jnp.float32)
    o_ref[0] = u[:, :128]
    o_ref[1] = u[:, 128:]


_tc_mid3 = pl.pallas_call(
    _tc_mid3_body,
    grid=(NP,),
    in_specs=[
        pl.BlockSpec((NC, R, 128), lambda i: (0, i, 0)),
        pl.BlockSpec((R, 128), lambda i: (i, 0)),
        pl.BlockSpec((R, 128), lambda i: (i, 0)),
        pl.BlockSpec((1, 128), lambda i: (0, 0)),
        pl.BlockSpec((128, 256), lambda i: (0, 0)),
    ],
    out_specs=pl.BlockSpec((NC, R, 128), lambda i: (0, i, 0)),
    out_shape=jax.ShapeDtypeStruct((NC, N, 128), jnp.float32),
)


def _tc_head_body(s_ref, h_ref, dinvb_ref, b3_ref, blo_ref, bhi_ref, cnt_ref,
                  wf1_ref, bf1_ref, wf2_ref, bf2_ref, wf3_ref, bf3_ref,
                  o_ref, acc):
    i = pl.program_id(0)

    @pl.when(i == 0)
    def _():
        acc[...] = jnp.zeros_like(acc)

    pre = jnp.concatenate([s_ref[0] + h_ref[0], s_ref[1] + h_ref[1]], axis=1)
    dinv = dinvb_ref[:, 0:1]
    h4 = jnp.maximum(dinv * pre + b3_ref[...], 0.0)
    ridx = lax.broadcasted_iota(jnp.int32, (R, B), 0) + i * R
    maskt = ((ridx >= blo_ref[...]) & (ridx < bhi_ref[...])).astype(jnp.float32)
    acc[...] += lax.dot_general(maskt, h4, (((0,), (0,)), ((), ())),
                                preferred_element_type=jnp.float32)

    @pl.when(i == NP - 1)
    def _():
        g = acc[...] / cnt_ref[...]
        g1 = jnp.maximum(
            jnp.dot(g, wf1_ref[...], preferred_element_type=jnp.float32)
            + bf1_ref[...], 0.0)
        g2 = jnp.maximum(
            jnp.dot(g1, wf2_ref[...], preferred_element_type=jnp.float32)
            + bf2_ref[...], 0.0)
        o_ref[...] = (jnp.dot(g2, wf3_ref[...],
                              preferred_element_type=jnp.float32)
                      + bf3_ref[...])


_tc_head = pl.pallas_call(
    _tc_head_body,
    grid=(NP,),
    in_specs=[
        pl.BlockSpec((NC, R, 128), lambda i: (0, i, 0)),
        pl.BlockSpec((NC, R, 128), lambda i: (0, i, 0)),
        pl.BlockSpec((R, 128), lambda i: (i, 0)),
        pl.BlockSpec((1, D), lambda i: (0, 0)),
        pl.BlockSpec((1, B), lambda i: (0, 0)),
        pl.BlockSpec((1, B), lambda i: (0, 0)),
        pl.BlockSpec((B, 1), lambda i: (0, 0)),
        pl.BlockSpec((D, 128), lambda i: (0, 0)),
        pl.BlockSpec((1, 128), lambda i: (0, 0)),
        pl.BlockSpec((128, 64), lambda i: (0, 0)),
        pl.BlockSpec((1, 64), lambda i: (0, 0)),
        pl.BlockSpec((64, 128), lambda i: (0, 0)),
        pl.BlockSpec((1, 128), lambda i: (0, 0)),
    ],
    out_specs=pl.BlockSpec((B, 128), lambda i: (0, 0)),
    out_shape=jax.ShapeDtypeStruct((B, 128), jnp.float32),
    scratch_shapes=[
        pltpu.VMEM((B, D), jnp.float32),
    ],
)


def _blockdiag4(a):
    r, c = a.shape
    out = jnp.zeros((4, r, 4, c), a.dtype)
    for j in range(4):
        out = out.at[j, :, j, :].set(a)
    return out.reshape(4 * r, 4 * c)


def kernel(x, edge_index, batch, W1, b1, W2, b2, W3, b3,
           Wf1, bf1, Wf2, bf2, Wf3, bf3):
    e3 = edge_index.reshape(2, NROWS, CH)
    x4 = x.reshape(NQ, 4 * D)

    bnd = jnp.sum(batch[:, None] < jnp.arange(B + 1, dtype=batch.dtype)[None, :],
                  axis=0, dtype=jnp.int32)
    blo = bnd[:B].reshape(1, B)
    bhi = bnd[1:].reshape(1, B)
    cnt = jnp.maximum(bhi - blo, 1).astype(jnp.float32).reshape(B, 1)

    w14_0 = _blockdiag4(W1[:, :32])
    w14_1 = _blockdiag4(W1[:, 32:])
    bd2_0 = _blockdiag4(W2[:32, :])
    bd2_1 = _blockdiag4(W2[32:, :])
    b4_0 = jnp.tile(b1[:32], 4).reshape(1, 128)
    b4_1 = jnp.tile(b1[32:], 4).reshape(1, 128)

    ones32 = jnp.ones((CH, DEGW), jnp.float32)
    zeros32 = jnp.zeros((NPS, DEGW), jnp.float32)
    z128 = jnp.zeros((NPS, 128), jnp.float32)

    degp = _deg_kernel()(e3, ones32, zeros32)
    degv = degp.reshape(NC, NQ, 128)

    h1, dinvq = _tc_first(x4, w14_0, w14_1, degv)
    dinvb = dinvq.reshape(N, 128)
    s1 = _edge_fsplit(32)(e3, h1.reshape(NC, N, 32), zeros32)

    h2q = _tc_mid2(s1.reshape(NC, NQ, 128), h1, degv, b4_0, b4_1, bd2_0, bd2_1)
    h2 = h2q.reshape(N, 128)
    s2 = _edge_esplit()(e3, h2, z128)

    h3 = _tc_mid3(s2, h2, dinvb, b2.reshape(1, 128), W3)
    s3 = _edge_fsplit(128)(e3, h3, z128)

    wf3p = jnp.pad(Wf3, ((0, 0), (0, 128 - Wf3.shape[1])))
    bf3p = jnp.pad(bf3, (0, 128 - bf3.shape[0])).reshape(1, 128)
    out = _tc_head(s3, h3, dinvb, b3.reshape(1, D),
                   blo, bhi, cnt,
                   Wf1, bf1.reshape(1, 128), Wf2, bf2.reshape(1, 64),
                   wf3p, bf3p)
    return out[:, :Wf3.shape[1]]


# speedup vs baseline: 19.8159x; 1.0544x over previous
import functools

import jax
import jax.numpy as jnp
from jax import lax
from jax.experimental import pallas as pl
from jax.experimental.pallas import tpu as pltpu
from jax.experimental.pallas import tpu_sc as plsc

N = 10000
E = 160000
D = 256
B = 8

NC = 2
NS = 16
CH = 125
NROWS = E // CH
RPS = NROWS // NS
SEG = 2
SROWS = RPS // SEG
RPW = NROWS // (NS * NC)
NPAD = 10240
NPS = NPAD // NS
NLAST = N - (NS - 1) * NPS
DEGW = 32

NQ = N // 4
RQ = 500
NPQ = NQ // RQ
R = 400
NP = N // R


def _deg_body(e_hbm, ones_hbm, zeros_hbm, out_hbm, idx_v, ones_v, acc, sem):
    c = lax.axis_index("c")
    s = lax.axis_index("s")
    pltpu.sync_copy(zeros_hbm, acc.at[pl.ds(s * NPS, NPS)])
    wid = c * NS + s
    pltpu.sync_copy(e_hbm.at[1].at[pl.ds(wid * RPW, RPW)], idx_v)
    pltpu.sync_copy(ones_hbm, ones_v)
    plsc.subcore_barrier()

    def chunk(j, carry):
        pltpu.sync_copy(ones_v, acc.at[idx_v.at[j]], add=True)
        return carry

    lax.fori_loop(0, RPW, chunk, 0)
    plsc.subcore_barrier()
    _copy_out(acc, out_hbm.at[c], s)


def _copy_out(acc, out2d, s):
    @pl.when(s < NS - 1)
    def _():
        pltpu.sync_copy(acc.at[pl.ds(s * NPS, NPS)],
                        out2d.at[pl.ds(s * NPS, NPS)])

    @pl.when(s == NS - 1)
    def _():
        pltpu.sync_copy(acc.at[pl.ds((NS - 1) * NPS, NLAST)],
                        out2d.at[pl.ds((NS - 1) * NPS, NLAST)])


@functools.cache
def _mesh():
    return plsc.VectorSubcoreMesh(core_axis_name="c", subcore_axis_name="s",
                                  num_cores=NC, num_subcores=NS)


_SC_PARAMS = pltpu.CompilerParams(use_tc_tiling_on_sc=False)


@functools.cache
def _deg_kernel():
    return pl.kernel(
        _deg_body,
        out_type=jax.ShapeDtypeStruct((NC, N, DEGW), jnp.float32),
        mesh=_mesh(),
        scratch_types=[
            pltpu.VMEM((RPW, CH), jnp.int32),
            pltpu.VMEM((CH, DEGW), jnp.float32),
            pltpu.VMEM_SHARED((NPAD, DEGW), jnp.float32),
            pltpu.SemaphoreType.DMA,
        ],
        compiler_params=_SC_PARAMS,
    )


def _edge_pipeline(e_hbm, tab, acc, src_v, dst_v, bufs, sems,
                   row_base, nrows, nseg):
    nb = len(bufs)

    def seg(t, carry):
        base = row_base + t * nrows
        pltpu.sync_copy(e_hbm.at[0].at[pl.ds(base, nrows)], src_v)
        pltpu.sync_copy(e_hbm.at[1].at[pl.ds(base, nrows)], dst_v)
        for b in range(nb):
            pltpu.async_copy(tab.at[src_v.at[b]], bufs[b], sems[b])

        def grp(i, carry2):
            g0 = nb * i
            for b in range(nb):
                g = g0 + b
                pltpu.make_async_copy(tab.at[src_v.at[g]], bufs[b],
                                      sems[b]).wait()
                pltpu.sync_copy(bufs[b], acc.at[dst_v.at[g]], add=True)

                @pl.when(g + nb < nrows)
                def _(g=g, b=b):
                    pltpu.async_copy(tab.at[src_v.at[g + nb]], bufs[b],
                                     sems[b])
            return carry2

        lax.fori_loop(0, nrows // nb, grp, 0)
        return carry

    lax.fori_loop(0, nseg, seg, 0)


def _edge_fsplit_body(fh, depth, e_hbm, hcat_hbm, zeros_hbm, out_hbm,
                      src_v, dst_v, *rest):
    bufs, sems, acc = rest[:depth], rest[depth + 1:], rest[depth]
    c = lax.axis_index("c")
    s = lax.axis_index("s")
    pltpu.sync_copy(zeros_hbm, acc.at[pl.ds(s * NPS, NPS)])
    plsc.subcore_barrier()
    _edge_pipeline(e_hbm, hcat_hbm.at[c], acc, src_v, dst_v, bufs, sems,
                   s * RPS, SROWS, SEG)
    plsc.subcore_barrier()
    _copy_out(acc, out_hbm.at[c], s)


def _edge_esplit_body(e_hbm, h_hbm, zeros_hbm, out_hbm,
                      src_v, dst_v, rows0, rows1, acc, sem0, sem1):
    c = lax.axis_index("c")
    s = lax.axis_index("s")
    pltpu.sync_copy(zeros_hbm, acc.at[pl.ds(s * NPS, NPS)])
    plsc.subcore_barrier()
    _edge_pipeline(e_hbm, h_hbm, acc, src_v, dst_v, [rows0, rows1],
                   [sem0, sem1], (c * NS + s) * RPW, RPW, 1)
    plsc.subcore_barrier()
    _copy_out(acc, out_hbm.at[c], s)


@functools.cache
def _edge_fsplit(fh):
    depth = 4 if fh <= 32 else 2
    return pl.kernel(
        functools.partial(_edge_fsplit_body, fh, depth),
        out_type=jax.ShapeDtypeStruct((NC, N, fh), jnp.float32),
        mesh=_mesh(),
        scratch_types=(
            [pltpu.VMEM((SROWS, CH), jnp.int32),
             pltpu.VMEM((SROWS, CH), jnp.int32)]
            + [pltpu.VMEM((CH, fh), jnp.float32)] * depth
            + [pltpu.VMEM_SHARED((NPAD, fh), jnp.float32)]
            + [pltpu.SemaphoreType.DMA] * depth),
        compiler_params=_SC_PARAMS,
    )


@functools.cache
def _edge_esplit():
    return pl.kernel(
        _edge_esplit_body,
        out_type=jax.ShapeDtypeStruct((NC, N, 128), jnp.float32),
        mesh=_mesh(),
        scratch_types=[
            pltpu.VMEM((RPW, CH), jnp.int32),
            pltpu.VMEM((RPW, CH), jnp.int32),
            pltpu.VMEM((CH, 128), jnp.float32),
            pltpu.VMEM((CH, 128), jnp.float32),
            pltpu.VMEM_SHARED((NPAD, 128), jnp.float32),
            pltpu.SemaphoreType.DMA,
            pltpu.SemaphoreType.DMA,
        ],
        compiler_params=_SC_PARAMS,
    )


def _tc_first_body(x_ref, w0_ref, w1_ref, degp_ref, o_ref, dinvb_ref):
    d = degp_ref[0] + degp_ref[1] + 1.0
    dinv4 = lax.rsqrt(d)
    xr = x_ref[...].reshape(NQ, 4, D)
    xj = [xr[:, j, :] for j in range(4)]
    for c, w_ref in ((0, w0_ref), (1, w1_ref)):
        w = w_ref[...]
        u = jnp.concatenate(
            [jnp.dot(xj[j], w, preferred_element_type=jnp.float32)
             for j in range(4)], axis=1)
        o_ref[c] = u * dinv4
    li = lax.broadcasted_iota(jnp.int32, (128, 512), 0)
    oi = lax.broadcasted_iota(jnp.int32, (128, 512), 1)
    m = ((li % 32 == 0) & (oi // 128 == li // 32)).astype(jnp.float32)
    dinvb_ref[...] = jnp.dot(dinv4, m, preferred_element_type=jnp.float32)


_tc_first = pl.pallas_call(
    _tc_first_body,
    out_shape=[
        jax.ShapeDtypeStruct((NC, NQ, 128), jnp.float32),
        jax.ShapeDtypeStruct((NQ, 512), jnp.float32),
    ],
)


def _tc_mid2_body(s_ref, h_ref, degp_ref, b0_ref, b1_ref, w0_ref, w1_ref,
                  o_ref):
    d = degp_ref[0] + degp_ref[1] + 1.0
    dinv4 = lax.rsqrt(d)
    hp0 = jnp.maximum(dinv4 * (s_ref[0] + h_ref[0]) + b0_ref[...], 0.0) * dinv4
    hp1 = jnp.maximum(dinv4 * (s_ref[1] + h_ref[1]) + b1_ref[...], 0.0) * dinv4
    o_ref[...] = (
        jnp.dot(hp0, w0_ref[...], preferred_element_type=jnp.float32)
        + jnp.dot(hp1, w1_ref[...], preferred_element_type=jnp.float32))


_tc_mid2 = pl.pallas_call(
    _tc_mid2_body,
    out_shape=jax.ShapeDtypeStruct((NQ, 512), jnp.float32),
)


def _tc_mid3_body(s_ref, h_ref, dinvb_ref, b_ref, w_ref, o_ref):
    dinv = dinvb_ref[...]
    pre = s_ref[0] + s_ref[1] + h_ref[...]
    h = jnp.maximum(dinv * pre + b_ref[...], 0.0) * dinv
    u = jnp.dot(h, w_ref[...], preferred_element_type=jnp.float32)
    o_ref[0] = u[:, :128]
    o_ref[1] = u[:, 128:]


_tc_mid3 = pl.pallas_call(
    _tc_mid3_body,
    grid=(NP,),
    in_specs=[
        pl.BlockSpec((NC, R, 128), lambda i: (0, i, 0)),
        pl.BlockSpec((R, 128), lambda i: (i, 0)),
        pl.BlockSpec((R, 128), lambda i: (i, 0)),
        pl.BlockSpec((1, 128), lambda i: (0, 0)),
        pl.BlockSpec((128, 256), lambda i: (0, 0)),
    ],
    out_specs=pl.BlockSpec((NC, R, 128), lambda i: (0, i, 0)),
    out_shape=jax.ShapeDtypeStruct((NC, N, 128), jnp.float32),
)


def _tc_head_body(s_ref, h_ref, dinvb_ref, b3_ref, blo_ref, bhi_ref, cnt_ref,
                  wf1_ref, bf1_ref, wf2_ref, bf2_ref, wf3_ref, bf3_ref,
                  o_ref, acc):
    i = pl.program_id(0)

    @pl.when(i == 0)
    def _():
        acc[...] = jnp.zeros_like(acc)

    pre = jnp.concatenate([s_ref[0] + h_ref[0], s_ref[1] + h_ref[1]], axis=1)
    dinv = dinvb_ref[:, 0:1]
    h4 = jnp.maximum(dinv * pre + b3_ref[...], 0.0)
    ridx = lax.broadcasted_iota(jnp.int32, (R, B), 0) + i * R
    maskt = ((ridx >= blo_ref[...]) & (ridx < bhi_ref[...])).astype(jnp.float32)
    acc[...] += lax.dot_general(maskt, h4, (((0,), (0,)), ((), ())),
                                preferred_element_type=jnp.float32)

    @pl.when(i == NP - 1)
    def _():
        g = acc[...] / cnt_ref[...]
        g1 = jnp.maximum(
            jnp.dot(g, wf1_ref[...], preferred_element_type=jnp.float32)
            + bf1_ref[...], 0.0)
        g2 = jnp.maximum(
            jnp.dot(g1, wf2_ref[...], preferred_element_type=jnp.float32)
            + bf2_ref[...], 0.0)
        o_ref[...] = (jnp.dot(g2, wf3_ref[...],
                              preferred_element_type=jnp.float32)
                      + bf3_ref[...])


_tc_head = pl.pallas_call(
    _tc_head_body,
    grid=(NP,),
    in_specs=[
        pl.BlockSpec((NC, R, 128), lambda i: (0, i, 0)),
        pl.BlockSpec((NC, R, 128), lambda i: (0, i, 0)),
        pl.BlockSpec((R, 128), lambda i: (i, 0)),
        pl.BlockSpec((1, D), lambda i: (0, 0)),
        pl.BlockSpec((1, B), lambda i: (0, 0)),
        pl.BlockSpec((1, B), lambda i: (0, 0)),
        pl.BlockSpec((B, 1), lambda i: (0, 0)),
        pl.BlockSpec((D, 128), lambda i: (0, 0)),
        pl.BlockSpec((1, 128), lambda i: (0, 0)),
        pl.BlockSpec((128, 64), lambda i: (0, 0)),
        pl.BlockSpec((1, 64), lambda i: (0, 0)),
        pl.BlockSpec((64, 128), lambda i: (0, 0)),
        pl.BlockSpec((1, 128), lambda i: (0, 0)),
    ],
    out_specs=pl.BlockSpec((B, 128), lambda i: (0, 0)),
    out_shape=jax.ShapeDtypeStruct((B, 128), jnp.float32),
    scratch_shapes=[
        pltpu.VMEM((B, D), jnp.float32),
    ],
)


def _blockdiag4(a):
    r, c = a.shape
    out = jnp.zeros((4, r, 4, c), a.dtype)
    for j in range(4):
        out = out.at[j, :, j, :].set(a)
    return out.reshape(4 * r, 4 * c)


def kernel(x, edge_index, batch, W1, b1, W2, b2, W3, b3,
           Wf1, bf1, Wf2, bf2, Wf3, bf3):
    e3 = edge_index.reshape(2, NROWS, CH)

    bnd = jnp.sum(batch[:, None] < jnp.arange(B + 1, dtype=batch.dtype)[None, :],
                  axis=0, dtype=jnp.int32)
    blo = bnd[:B].reshape(1, B)
    bhi = bnd[1:].reshape(1, B)
    cnt = jnp.maximum(bhi - blo, 1).astype(jnp.float32).reshape(B, 1)

    bd2_0 = _blockdiag4(W2[:32, :])
    bd2_1 = _blockdiag4(W2[32:, :])
    b4_0 = jnp.tile(b1[:32], 4).reshape(1, 128)
    b4_1 = jnp.tile(b1[32:], 4).reshape(1, 128)

    ones32 = jnp.ones((CH, DEGW), jnp.float32)
    zeros32 = jnp.zeros((NPS, DEGW), jnp.float32)
    z128 = jnp.zeros((NPS, 128), jnp.float32)

    degp = _deg_kernel()(e3, ones32, zeros32)
    degv = degp.reshape(NC, NQ, 128)

    h1, dinvq = _tc_first(x, W1[:, :32], W1[:, 32:], degv)
    dinvb = dinvq.reshape(N, 128)
    s1 = _edge_fsplit(32)(e3, h1.reshape(NC, N, 32), zeros32)

    h2q = _tc_mid2(s1.reshape(NC, NQ, 128), h1, degv, b4_0, b4_1, bd2_0, bd2_1)
    h2 = h2q.reshape(N, 128)
    s2 = _edge_esplit()(e3, h2, z128)

    h3 = _tc_mid3(s2, h2, dinvb, b2.reshape(1, 128), W3)
    s3 = _edge_fsplit(128)(e3, h3, z128)

    wf3p = jnp.pad(Wf3, ((0, 0), (0, 128 - Wf3.shape[1])))
    bf3p = jnp.pad(bf3, (0, 128 - bf3.shape[0])).reshape(1, 128)
    out = _tc_head(s3, h3, dinvb, b3.reshape(1, D),
                   blo, bhi, cnt,
                   Wf1, bf1.reshape(1, 128), Wf2, bf2.reshape(1, 64),
                   wf3p, bf3p)
    return out[:, :Wf3.shape[1]]
